# Initial kernel scaffold; baseline (speedup 1.0000x reference)
import jax
import jax.numpy as jnp
from jax.experimental import pallas as pl


def kernel(edge_index, user_emb, item_emb):
    raise NotImplementedError("write your pallas kernel here")


# speedup vs baseline: 5.0591x; 5.0591x over previous
import functools

import numpy as np

import jax
import jax.numpy as jnp
from jax import lax
from jax.experimental import pallas as pl
from jax.experimental.pallas import tpu as pltpu
from jax.experimental.pallas import tpu_sc as plsc

_N_USER = 5000
_N_ITEM = 5000
_N = _N_USER + _N_ITEM
_E = 320000
_D = 128
_F = 4
_BW = _D // _F
_LAYER = 2
_ITER = 2

_NC = 2
_NS = 16
_NW = _NC * _NS
_B = 128
_TOTBLK = _E // _B
_FULL = _TOTBLK // _NW
_REM = _TOTBLK - _FULL * _NW

_CHUNK = 640
_LAST_CHUNK = _N - 15 * _CHUNK

_NP = 10112
_LAST_CHUNK_P = _NP - 15 * _CHUNK

_mesh = plsc.VectorSubcoreMesh(core_axis_name="c", subcore_axis_name="s")

def _block_scale(x, col4):
    return jnp.concatenate(
        [x[:, f * _BW:(f + 1) * _BW] * col4[:, f:f + 1] for f in range(_F)],
        axis=1)


def _block_inv_norm(x):
    ss = jnp.concatenate(
        [jnp.sum(x[:, f * _BW:(f + 1) * _BW] ** 2, axis=1, keepdims=True)
         for f in range(_F)], axis=1)
    return 1.0 / jnp.maximum(jnp.sqrt(ss), 1e-12)


def _node_chunk_copy(s, fn):
    @pl.when(s < _NS - 1)
    def _():
        fn(s * _CHUNK, _CHUNK)

    @pl.when(s == _NS - 1)
    def _():
        fn((_NS - 1) * _CHUNK, _LAST_CHUNK)


def _node_chunk_copy_p(s, fn):
    @pl.when(s < _NS - 1)
    def _():
        fn(s * _CHUNK, _CHUNK)

    @pl.when(s == _NS - 1)
    def _():
        fn((_NS - 1) * _CHUNK, _LAST_CHUNK_P)


@functools.partial(
    pl.kernel,
    out_type=[
        jax.ShapeDtypeStruct((_F, _E), jnp.float32),
        jax.ShapeDtypeStruct((_NC, _F, _NP), jnp.float32),
    ],
    mesh=_mesh,
    compiler_params=pltpu.CompilerParams(needs_layout_passes=False),
    scratch_types=[
        pltpu.VMEM((_F, _B), jnp.float32),
        pltpu.VMEM((_F, _B), jnp.float32),
        pltpu.VMEM((_B,), jnp.int32),
        [pltpu.VMEM_SHARED((_NP,), jnp.float32) for _ in range(_F)],
    ],
)
def _deg_sc(fv_hbm, head_hbm, zeros_hbm, scores_hbm, degp_hbm,
            fv_v, sc_v, hi_v, deg_sh):
    c = lax.axis_index("c")
    s = lax.axis_index("s")
    wid = s * _NC + c

    def zero(r0, nr):
        for f in range(_F):
            pltpu.sync_copy(zeros_hbm.at[f, pl.ds(r0, nr)],
                            deg_sh[f].at[pl.ds(r0, nr)])
    _node_chunk_copy_p(s, zero)
    plsc.subcore_barrier()

    def block(i, carry):
        base = (i * _NW + wid) * _B
        pltpu.sync_copy(head_hbm.at[pl.ds(base, _B)], hi_v)
        pltpu.sync_copy(fv_hbm.at[:, pl.ds(base, _B)], fv_v)
        for g in range(_B // 16):
            sl = pl.ds(g * 16, 16)
            v = [fv_v[f, sl] for f in range(_F)]
            m = jnp.maximum(jnp.maximum(v[0], v[1]), jnp.maximum(v[2], v[3]))
            ex = [jnp.exp(v[f] - m) for f in range(_F)]
            inv = 1.0 / ((ex[0] + ex[1]) + (ex[2] + ex[3]))
            for f in range(_F):
                sc_v[f, sl] = ex[f] * inv
        pltpu.sync_copy(sc_v, scores_hbm.at[:, pl.ds(base, _B)])
        for f in range(_F):
            pltpu.sync_copy(sc_v.at[f], deg_sh[f].at[hi_v], add=True)
        return carry

    lax.fori_loop(0, _FULL + (wid < _REM).astype(jnp.int32), block, 0)
    plsc.subcore_barrier()

    def out(r0, nr):
        for f in range(_F):
            pltpu.sync_copy(deg_sh[f].at[pl.ds(r0, nr)],
                            degp_hbm.at[c, f, pl.ds(r0, nr)])
    _node_chunk_copy_p(s, out)


@functools.partial(
    pl.kernel,
    out_type=jax.ShapeDtypeStruct((_NC, _N, _D), jnp.float32),
    mesh=_mesh,
    compiler_params=pltpu.CompilerParams(needs_layout_passes=False),
    scratch_types=[
        pltpu.VMEM((_B, _D), jnp.float32),
        pltpu.VMEM((_F, _B), jnp.float32),
        pltpu.VMEM((_B,), jnp.int32),
        pltpu.VMEM((_B,), jnp.int32),
        pltpu.VMEM_SHARED((_N, _D), jnp.float32),
        pltpu.SemaphoreType.DMA,
    ],
)
def _spmm_sc(xpre_hbm, head_hbm, tail_hbm, scores_hbm, zeros_hbm, zp_hbm,
             rows_v, sc_v, hi_v, ti_v, z_sh, sem):
    c = lax.axis_index("c")
    s = lax.axis_index("s")
    wid = s * _NC + c

    def zero(r0, nr):
        pltpu.sync_copy(zeros_hbm.at[pl.ds(r0, nr), :], z_sh.at[pl.ds(r0, nr), :])
    _node_chunk_copy(s, zero)
    plsc.subcore_barrier()

    def block(i, carry):
        base = (i * _NW + wid) * _B
        pltpu.sync_copy(tail_hbm.at[pl.ds(base, _B)], ti_v)
        pltpu.sync_copy(head_hbm.at[pl.ds(base, _B)], hi_v)
        pltpu.sync_copy(scores_hbm.at[:, pl.ds(base, _B)], sc_v)
        pltpu.async_copy(xpre_hbm.at[ti_v], rows_v, sem).wait()
        for g in range(_B // 16):
            svs = [sc_v[f, pl.ds(g * 16, 16)] for f in range(_F)]
            for j in range(16):
                e = g * 16 + j
                for f in range(_F):
                    sf = svs[f][j]
                    for h in range(2):
                        sl = pl.ds(f * _BW + h * 16, 16)
                        rows_v[e, sl] = rows_v[e, sl] * sf
        pltpu.sync_copy(rows_v, z_sh.at[hi_v], add=True)
        return carry

    lax.fori_loop(0, _FULL + (wid < _REM).astype(jnp.int32), block, 0)
    plsc.subcore_barrier()

    def out(r0, nr):
        pltpu.sync_copy(z_sh.at[pl.ds(r0, nr), :], zp_hbm.at[c, pl.ds(r0, nr), :])
    _node_chunk_copy(s, out)


@functools.partial(
    pl.kernel,
    out_type=jax.ShapeDtypeStruct((_F, _E), jnp.float32),
    mesh=_mesh,
    compiler_params=pltpu.CompilerParams(needs_layout_passes=False),
    scratch_types=[
        pltpu.VMEM((_B, _D), jnp.float32),
        pltpu.VMEM((_B, _D), jnp.float32),
        pltpu.VMEM((_F, _B), jnp.float32),
        pltpu.VMEM((_F, _B), jnp.float32),
        pltpu.VMEM((_B,), jnp.int32),
        pltpu.VMEM((_B,), jnp.int32),
        pltpu.SemaphoreType.DMA,
        pltpu.SemaphoreType.DMA,
    ],
)
def _rout_sc(hpre_hbm, tpre_hbm, fv_hbm, head_hbm, tail_hbm, fv_out_hbm,
             hb_v, tb_v, fv_v, out_v, hi_v, ti_v, sem1, sem2):
    c = lax.axis_index("c")
    s = lax.axis_index("s")
    wid = s * _NC + c

    def block(i, carry):
        base = (i * _NW + wid) * _B
        pltpu.sync_copy(head_hbm.at[pl.ds(base, _B)], hi_v)
        pltpu.sync_copy(tail_hbm.at[pl.ds(base, _B)], ti_v)
        pltpu.sync_copy(fv_hbm.at[:, pl.ds(base, _B)], fv_v)
        d1 = pltpu.async_copy(hpre_hbm.at[hi_v], hb_v, sem1)
        d2 = pltpu.async_copy(tpre_hbm.at[ti_v], tb_v, sem2)
        d1.wait()
        d2.wait()
        def group(g, carry2):
            gsl = pl.ds(g * 16, 16)
            ridx = lax.iota(jnp.int32, 16) + g * 16
            acc = [fv_v[f, gsl] for f in range(_F)]
            for j in range(_D):
                cidx = jnp.full((16,), j, jnp.int32)
                hv = plsc.load_gather(hb_v, [ridx, cidx])
                tv = plsc.load_gather(tb_v, [ridx, cidx])
                acc[j // _BW] = acc[j // _BW] + hv * tv
            for f in range(_F):
                out_v[f, gsl] = acc[f]
            return carry2

        lax.fori_loop(0, _B // 16, group, 0)
        pltpu.sync_copy(out_v, fv_out_hbm.at[:, pl.ds(base, _B)])
        return carry

    lax.fori_loop(0, _FULL + (wid < _REM).astype(jnp.int32), block, 0)


def _prep_tc(degp_ref, ego_ref, xpre_ref, dcolt_ref):
    deg = degp_ref[0, :, :_N] + degp_ref[1, :, :_N]
    dcol = lax.rsqrt(deg).T
    dcolt_ref[...] = dcol
    xpre_ref[...] = _block_scale(ego_ref[...], dcol)


def _post_tc(zp_ref, dcolt_ref, y_ref, hpre_ref):
    z = zp_ref[0, :, :] + zp_ref[1, :, :]
    y_ref[...] = _block_scale(z, dcolt_ref[...])
    hpre_ref[...] = _block_scale(z, _block_inv_norm(z))


def _tpre_tc(ego_ref, tpre_ref):
    ego = ego_ref[...]
    tpre_ref[...] = jnp.tanh(_block_scale(ego, _block_inv_norm(ego)))


def _mean_tc(e0_ref, e1_ref, e2_ref, out_ref):
    out_ref[...] = (e0_ref[...] + e1_ref[...] + e2_ref[...]) * (1.0 / 3.0)


_prep_call = pl.pallas_call(
    _prep_tc, out_shape=[jax.ShapeDtypeStruct((_N, _D), jnp.float32),
                         jax.ShapeDtypeStruct((_N, _F), jnp.float32)])
_BN = 2000
_post_call = pl.pallas_call(
    _post_tc,
    grid=(_N // _BN,),
    in_specs=[pl.BlockSpec((_NC, _BN, _D), lambda i: (0, i, 0)),
              pl.BlockSpec((_BN, _F), lambda i: (i, 0))],
    out_specs=[pl.BlockSpec((_BN, _D), lambda i: (i, 0)),
               pl.BlockSpec((_BN, _D), lambda i: (i, 0))],
    out_shape=[jax.ShapeDtypeStruct((_N, _D), jnp.float32),
               jax.ShapeDtypeStruct((_N, _D), jnp.float32)])
_tpre_call = pl.pallas_call(
    _tpre_tc, out_shape=jax.ShapeDtypeStruct((_N, _D), jnp.float32))
_mean_call = pl.pallas_call(
    _mean_tc, out_shape=jax.ShapeDtypeStruct((_N, _D), jnp.float32))


def kernel(edge_index, user_emb, item_emb):
    head = edge_index[0].astype(jnp.int32)
    tail = edge_index[1].astype(jnp.int32)
    all_emb = jnp.concatenate([user_emb, item_emb], axis=0)
    fv = jnp.ones((_F, _E), jnp.float32)
    zeros_4n = jnp.zeros((_F, _NP), jnp.float32)
    zeros_nd = jnp.zeros((_N, _D), jnp.float32)

    embs = [all_emb]
    for l in range(_LAYER):
        ego = all_emb
        tpre = _tpre_call(ego)
        y = None
        for t in range(_ITER):
            scores, degp = _deg_sc(fv, head, zeros_4n)
            xpre, dcolt = _prep_call(degp, ego)
            zp = _spmm_sc(xpre, head, tail, scores, zeros_nd)
            y, hpre = _post_call(zp, dcolt)
            if not (l == _LAYER - 1 and t == _ITER - 1):
                fv = _rout_sc(hpre, tpre, fv, head, tail)
        all_emb = y
        embs.append(all_emb)

    out = _mean_call(embs[0], embs[1], embs[2])
    return out[:_N_USER], out[_N_USER:]


# speedup vs baseline: 5.8442x; 1.1552x over previous
import functools

import numpy as np

import jax
import jax.numpy as jnp
from jax import lax
from jax.experimental import pallas as pl
from jax.experimental.pallas import tpu as pltpu
from jax.experimental.pallas import tpu_sc as plsc

_N_USER = 5000
_N_ITEM = 5000
_N = _N_USER + _N_ITEM
_E = 320000
_D = 128
_F = 4
_BW = _D // _F
_LAYER = 2
_ITER = 2

_NC = 2
_NS = 16
_NW = _NC * _NS
_B = 128
_TOTBLK = _E // _B
_FULL = _TOTBLK // _NW
_REM = _TOTBLK - _FULL * _NW

_CHUNK = 640
_LAST_CHUNK = _N - 15 * _CHUNK

_NP = 10112
_LAST_CHUNK_P = _NP - 15 * _CHUNK

_mesh = plsc.VectorSubcoreMesh(core_axis_name="c", subcore_axis_name="s")

def _block_scale(x, col4):
    return jnp.concatenate(
        [x[:, f * _BW:(f + 1) * _BW] * col4[:, f:f + 1] for f in range(_F)],
        axis=1)


def _block_inv_norm(x):
    ss = jnp.concatenate(
        [jnp.sum(x[:, f * _BW:(f + 1) * _BW] ** 2, axis=1, keepdims=True)
         for f in range(_F)], axis=1)
    return 1.0 / jnp.maximum(jnp.sqrt(ss), 1e-12)


def _node_chunk_copy(s, fn):
    @pl.when(s < _NS - 1)
    def _():
        fn(s * _CHUNK, _CHUNK)

    @pl.when(s == _NS - 1)
    def _():
        fn((_NS - 1) * _CHUNK, _LAST_CHUNK)


def _node_chunk_copy_p(s, fn):
    @pl.when(s < _NS - 1)
    def _():
        fn(s * _CHUNK, _CHUNK)

    @pl.when(s == _NS - 1)
    def _():
        fn((_NS - 1) * _CHUNK, _LAST_CHUNK_P)


@functools.partial(
    pl.kernel,
    out_type=[
        jax.ShapeDtypeStruct((_F, _E), jnp.float32),
        jax.ShapeDtypeStruct((_NC, _F, _NP), jnp.float32),
    ],
    mesh=_mesh,
    compiler_params=pltpu.CompilerParams(needs_layout_passes=False),
    scratch_types=[
        pltpu.VMEM((_F, _B), jnp.float32),
        pltpu.VMEM((_F, _B), jnp.float32),
        pltpu.VMEM((_B,), jnp.int32),
        [pltpu.VMEM_SHARED((_NP,), jnp.float32) for _ in range(_F)],
    ],
)
def _deg_sc(fv_hbm, head_hbm, zeros_hbm, scores_hbm, degp_hbm,
            fv_v, sc_v, hi_v, deg_sh):
    c = lax.axis_index("c")
    s = lax.axis_index("s")
    wid = s * _NC + c

    def zero(r0, nr):
        for f in range(_F):
            pltpu.sync_copy(zeros_hbm.at[f, pl.ds(r0, nr)],
                            deg_sh[f].at[pl.ds(r0, nr)])
    _node_chunk_copy_p(s, zero)
    plsc.subcore_barrier()

    def block(i, carry):
        base = (i * _NW + wid) * _B
        pltpu.sync_copy(head_hbm.at[pl.ds(base, _B)], hi_v)
        pltpu.sync_copy(fv_hbm.at[:, pl.ds(base, _B)], fv_v)
        for g in range(_B // 16):
            sl = pl.ds(g * 16, 16)
            v = [fv_v[f, sl] for f in range(_F)]
            m = jnp.maximum(jnp.maximum(v[0], v[1]), jnp.maximum(v[2], v[3]))
            ex = [jnp.exp(v[f] - m) for f in range(_F)]
            inv = 1.0 / ((ex[0] + ex[1]) + (ex[2] + ex[3]))
            for f in range(_F):
                sc_v[f, sl] = ex[f] * inv
        pltpu.sync_copy(sc_v, scores_hbm.at[:, pl.ds(base, _B)])
        for f in range(_F):
            pltpu.sync_copy(sc_v.at[f], deg_sh[f].at[hi_v], add=True)
        return carry

    lax.fori_loop(0, _FULL + (wid < _REM).astype(jnp.int32), block, 0)
    plsc.subcore_barrier()

    def out(r0, nr):
        for f in range(_F):
            pltpu.sync_copy(deg_sh[f].at[pl.ds(r0, nr)],
                            degp_hbm.at[c, f, pl.ds(r0, nr)])
    _node_chunk_copy_p(s, out)


@functools.partial(
    pl.kernel,
    out_type=jax.ShapeDtypeStruct((_NC, _N, _D), jnp.float32),
    mesh=_mesh,
    compiler_params=pltpu.CompilerParams(needs_layout_passes=False),
    scratch_types=[
        pltpu.VMEM((_B, _D), jnp.float32),
        pltpu.VMEM((_F, _B), jnp.float32),
        pltpu.VMEM((_B,), jnp.int32),
        pltpu.VMEM((_B,), jnp.int32),
        pltpu.VMEM_SHARED((_N, _D), jnp.float32),
        pltpu.SemaphoreType.DMA,
    ],
)
def _spmm_sc(xpre_hbm, head_hbm, tail_hbm, scores_hbm, zeros_hbm, zp_hbm,
             rows_v, sc_v, hi_v, ti_v, z_sh, sem):
    c = lax.axis_index("c")
    s = lax.axis_index("s")
    wid = s * _NC + c

    def zero(r0, nr):
        pltpu.sync_copy(zeros_hbm.at[pl.ds(r0, nr), :], z_sh.at[pl.ds(r0, nr), :])
    _node_chunk_copy(s, zero)
    plsc.subcore_barrier()

    def block(i, carry):
        base = (i * _NW + wid) * _B
        pltpu.sync_copy(tail_hbm.at[pl.ds(base, _B)], ti_v)
        pltpu.sync_copy(head_hbm.at[pl.ds(base, _B)], hi_v)
        pltpu.sync_copy(scores_hbm.at[:, pl.ds(base, _B)], sc_v)
        pltpu.async_copy(xpre_hbm.at[ti_v], rows_v, sem).wait()
        for g in range(_B // 16):
            iota = lax.iota(jnp.int32, 16)
            ridx = iota + g * 16
            svs = [sc_v[f, pl.ds(g * 16, 16)] for f in range(_F)]

            def scale_chunk(jj, carry2, f):
                for dj in range(8):
                    j = jj * 8 + dj
                    cidx = (f * _BW) + ((iota + j) & (_BW - 1))
                    v = plsc.load_gather(rows_v, [ridx, cidx]) * svs[f]
                    plsc.store_scatter(rows_v, [ridx, cidx], v)
                return carry2

            for f in range(_F):
                lax.fori_loop(0, _BW // 8,
                              functools.partial(scale_chunk, f=f), 0)
        pltpu.sync_copy(rows_v, z_sh.at[hi_v], add=True)
        return carry

    lax.fori_loop(0, _FULL + (wid < _REM).astype(jnp.int32), block, 0)
    plsc.subcore_barrier()

    def out(r0, nr):
        pltpu.sync_copy(z_sh.at[pl.ds(r0, nr), :], zp_hbm.at[c, pl.ds(r0, nr), :])
    _node_chunk_copy(s, out)


@functools.partial(
    pl.kernel,
    out_type=jax.ShapeDtypeStruct((_F, _E), jnp.float32),
    mesh=_mesh,
    compiler_params=pltpu.CompilerParams(needs_layout_passes=False),
    scratch_types=[
        pltpu.VMEM((_B, _D), jnp.float32),
        pltpu.VMEM((_B, _D), jnp.float32),
        pltpu.VMEM((_F, _B), jnp.float32),
        pltpu.VMEM((_F, _B), jnp.float32),
        pltpu.VMEM((_B,), jnp.int32),
        pltpu.VMEM((_B,), jnp.int32),
        pltpu.SemaphoreType.DMA,
        pltpu.SemaphoreType.DMA,
    ],
)
def _rout_sc(hpre_hbm, tpre_hbm, fv_hbm, head_hbm, tail_hbm, fv_out_hbm,
             hb_v, tb_v, fv_v, out_v, hi_v, ti_v, sem1, sem2):
    c = lax.axis_index("c")
    s = lax.axis_index("s")
    wid = s * _NC + c

    def block(i, carry):
        base = (i * _NW + wid) * _B
        pltpu.sync_copy(head_hbm.at[pl.ds(base, _B)], hi_v)
        pltpu.sync_copy(tail_hbm.at[pl.ds(base, _B)], ti_v)
        pltpu.sync_copy(fv_hbm.at[:, pl.ds(base, _B)], fv_v)
        d1 = pltpu.async_copy(hpre_hbm.at[hi_v], hb_v, sem1)
        d2 = pltpu.async_copy(tpre_hbm.at[ti_v], tb_v, sem2)
        d1.wait()
        d2.wait()
        def group(g, carry2):
            gsl = pl.ds(g * 16, 16)
            iota = lax.iota(jnp.int32, 16)
            ridx = iota + g * 16

            def dot_chunk(jj, a, f):
                for dj in range(8):
                    j = jj * 8 + dj
                    cidx = (f * _BW) + ((iota + j) & (_BW - 1))
                    hv = plsc.load_gather(hb_v, [ridx, cidx])
                    tv = plsc.load_gather(tb_v, [ridx, cidx])
                    a = a + hv * tv
                return a

            for f in range(_F):
                acc = lax.fori_loop(0, _BW // 8,
                                    functools.partial(dot_chunk, f=f),
                                    fv_v[f, gsl])
                out_v[f, gsl] = acc
            return carry2

        lax.fori_loop(0, _B // 16, group, 0)
        pltpu.sync_copy(out_v, fv_out_hbm.at[:, pl.ds(base, _B)])
        return carry

    lax.fori_loop(0, _FULL + (wid < _REM).astype(jnp.int32), block, 0)


def _prep_tc(degp_ref, ego_ref, xpre_ref, dcolt_ref):
    deg = degp_ref[0, :, :_N] + degp_ref[1, :, :_N]
    dcol = lax.rsqrt(deg).T
    dcolt_ref[...] = dcol
    xpre_ref[...] = _block_scale(ego_ref[...], dcol)


def _post_tc(zp_ref, dcolt_ref, y_ref, hpre_ref):
    z = zp_ref[0, :, :] + zp_ref[1, :, :]
    y_ref[...] = _block_scale(z, dcolt_ref[...])
    hpre_ref[...] = _block_scale(z, _block_inv_norm(z))


def _tpre_tc(ego_ref, tpre_ref):
    ego = ego_ref[...]
    tpre_ref[...] = jnp.tanh(_block_scale(ego, _block_inv_norm(ego)))


def _mean_tc(e0_ref, e1_ref, e2_ref, out_ref):
    out_ref[...] = (e0_ref[...] + e1_ref[...] + e2_ref[...]) * (1.0 / 3.0)


_prep_call = pl.pallas_call(
    _prep_tc, out_shape=[jax.ShapeDtypeStruct((_N, _D), jnp.float32),
                         jax.ShapeDtypeStruct((_N, _F), jnp.float32)])
_BN = 2000
_post_call = pl.pallas_call(
    _post_tc,
    grid=(_N // _BN,),
    in_specs=[pl.BlockSpec((_NC, _BN, _D), lambda i: (0, i, 0)),
              pl.BlockSpec((_BN, _F), lambda i: (i, 0))],
    out_specs=[pl.BlockSpec((_BN, _D), lambda i: (i, 0)),
               pl.BlockSpec((_BN, _D), lambda i: (i, 0))],
    out_shape=[jax.ShapeDtypeStruct((_N, _D), jnp.float32),
               jax.ShapeDtypeStruct((_N, _D), jnp.float32)])
_tpre_call = pl.pallas_call(
    _tpre_tc, out_shape=jax.ShapeDtypeStruct((_N, _D), jnp.float32))
_mean_call = pl.pallas_call(
    _mean_tc, out_shape=jax.ShapeDtypeStruct((_N, _D), jnp.float32))


def kernel(edge_index, user_emb, item_emb):
    head = edge_index[0].astype(jnp.int32)
    tail = edge_index[1].astype(jnp.int32)
    all_emb = jnp.concatenate([user_emb, item_emb], axis=0)
    fv = jnp.ones((_F, _E), jnp.float32)
    zeros_4n = jnp.zeros((_F, _NP), jnp.float32)
    zeros_nd = jnp.zeros((_N, _D), jnp.float32)

    embs = [all_emb]
    for l in range(_LAYER):
        ego = all_emb
        tpre = _tpre_call(ego)
        y = None
        for t in range(_ITER):
            scores, degp = _deg_sc(fv, head, zeros_4n)
            xpre, dcolt = _prep_call(degp, ego)
            zp = _spmm_sc(xpre, head, tail, scores, zeros_nd)
            y, hpre = _post_call(zp, dcolt)
            if not (l == _LAYER - 1 and t == _ITER - 1):
                fv = _rout_sc(hpre, tpre, fv, head, tail)
        all_emb = y
        embs.append(all_emb)

    out = _mean_call(embs[0], embs[1], embs[2])
    return out[:_N_USER], out[_N_USER:]


# speedup vs baseline: 10.3203x; 1.7659x over previous
import functools

import numpy as np

import jax
import jax.numpy as jnp
from jax import lax
from jax.experimental import pallas as pl
from jax.experimental.pallas import tpu as pltpu
from jax.experimental.pallas import tpu_sc as plsc

_N_USER = 5000
_N_ITEM = 5000
_N = _N_USER + _N_ITEM
_E = 320000
_D = 128
_F = 4
_BW = _D // _F
_LAYER = 2
_ITER = 2

_NC = 2
_NS = 16
_NW = _NC * _NS
_B = 128
_TOTBLK = _E // _B
_FULL = _TOTBLK // _NW
_REM = _TOTBLK - _FULL * _NW

_CHUNK = 640
_LAST_CHUNK = _N - 15 * _CHUNK

_NP = 10112
_LAST_CHUNK_P = _NP - 15 * _CHUNK

_mesh = plsc.VectorSubcoreMesh(core_axis_name="c", subcore_axis_name="s")

def _block_scale(x, col4):
    return jnp.concatenate(
        [x[:, f * _BW:(f + 1) * _BW] * col4[:, f:f + 1] for f in range(_F)],
        axis=1)


def _block_inv_norm(x):
    ss = jnp.concatenate(
        [jnp.sum(x[:, f * _BW:(f + 1) * _BW] ** 2, axis=1, keepdims=True)
         for f in range(_F)], axis=1)
    return 1.0 / jnp.maximum(jnp.sqrt(ss), 1e-12)


def _node_chunk_copy(s, fn):
    @pl.when(s < _NS - 1)
    def _():
        fn(s * _CHUNK, _CHUNK)

    @pl.when(s == _NS - 1)
    def _():
        fn((_NS - 1) * _CHUNK, _LAST_CHUNK)


def _node_chunk_copy_p(s, fn):
    @pl.when(s < _NS - 1)
    def _():
        fn(s * _CHUNK, _CHUNK)

    @pl.when(s == _NS - 1)
    def _():
        fn((_NS - 1) * _CHUNK, _LAST_CHUNK_P)


@functools.partial(
    pl.kernel,
    out_type=[
        jax.ShapeDtypeStruct((_F, _E), jnp.float32),
        jax.ShapeDtypeStruct((_NC, _F, _NP), jnp.float32),
    ],
    mesh=_mesh,
    compiler_params=pltpu.CompilerParams(needs_layout_passes=False),
    scratch_types=[
        pltpu.VMEM((_F, _B), jnp.float32),
        pltpu.VMEM((_F, _B), jnp.float32),
        pltpu.VMEM((_B,), jnp.int32),
        [pltpu.VMEM_SHARED((_NP,), jnp.float32) for _ in range(_F)],
    ],
)
def _deg_sc(fv_hbm, head_hbm, zeros_hbm, scores_hbm, degp_hbm,
            fv_v, sc_v, hi_v, deg_sh):
    c = lax.axis_index("c")
    s = lax.axis_index("s")
    wid = s * _NC + c

    def zero(r0, nr):
        for f in range(_F):
            pltpu.sync_copy(zeros_hbm.at[f, pl.ds(r0, nr)],
                            deg_sh[f].at[pl.ds(r0, nr)])
    _node_chunk_copy_p(s, zero)
    plsc.subcore_barrier()

    def block(i, carry):
        base = (i * _NW + wid) * _B
        pltpu.sync_copy(head_hbm.at[pl.ds(base, _B)], hi_v)
        pltpu.sync_copy(fv_hbm.at[:, pl.ds(base, _B)], fv_v)
        for g in range(_B // 16):
            sl = pl.ds(g * 16, 16)
            v = [fv_v[f, sl] for f in range(_F)]
            m = jnp.maximum(jnp.maximum(v[0], v[1]), jnp.maximum(v[2], v[3]))
            ex = [jnp.exp(v[f] - m) for f in range(_F)]
            inv = 1.0 / ((ex[0] + ex[1]) + (ex[2] + ex[3]))
            for f in range(_F):
                sc_v[f, sl] = ex[f] * inv
        pltpu.sync_copy(sc_v, scores_hbm.at[:, pl.ds(base, _B)])
        for f in range(_F):
            pltpu.sync_copy(sc_v.at[f], deg_sh[f].at[hi_v], add=True)
        return carry

    lax.fori_loop(0, _FULL + (wid < _REM).astype(jnp.int32), block, 0)
    plsc.subcore_barrier()

    def out(r0, nr):
        for f in range(_F):
            pltpu.sync_copy(deg_sh[f].at[pl.ds(r0, nr)],
                            degp_hbm.at[c, f, pl.ds(r0, nr)])
    _node_chunk_copy_p(s, out)


@functools.partial(
    pl.kernel,
    out_type=jax.ShapeDtypeStruct((_NC, _N, _D), jnp.float32),
    mesh=_mesh,
    compiler_params=pltpu.CompilerParams(needs_layout_passes=False),
    scratch_types=[
        pltpu.VMEM((_B, _D), jnp.float32),
        pltpu.VMEM((_F, _B), jnp.float32),
        pltpu.VMEM((_B,), jnp.int32),
        pltpu.VMEM((_B,), jnp.int32),
        pltpu.VMEM_SHARED((_N, _D), jnp.float32),
        pltpu.SemaphoreType.DMA,
    ],
)
def _spmm_sc(xpre_hbm, head_hbm, tail_hbm, scores_hbm, zeros_hbm, zp_hbm,
             rows_v, sc_v, hi_v, ti_v, z_sh, sem):
    c = lax.axis_index("c")
    s = lax.axis_index("s")
    wid = s * _NC + c

    def zero(r0, nr):
        pltpu.sync_copy(zeros_hbm.at[pl.ds(r0, nr), :], z_sh.at[pl.ds(r0, nr), :])
    _node_chunk_copy(s, zero)
    plsc.subcore_barrier()

    def block(i, carry):
        base = (i * _NW + wid) * _B
        pltpu.sync_copy(tail_hbm.at[pl.ds(base, _B)], ti_v)
        pltpu.sync_copy(head_hbm.at[pl.ds(base, _B)], hi_v)
        pltpu.sync_copy(scores_hbm.at[:, pl.ds(base, _B)], sc_v)
        pltpu.async_copy(xpre_hbm.at[ti_v], rows_v, sem).wait()
        for g in range(_B // 16):
            svs = [sc_v[f, pl.ds(g * 16, 16)] for f in range(_F)]
            for j in range(16):
                e = g * 16 + j
                for f in range(_F):
                    sf = svs[f][j]
                    for h in range(2):
                        sl = pl.ds(f * _BW + h * 16, 16)
                        rows_v[e, sl] = rows_v[e, sl] * sf
        pltpu.sync_copy(rows_v, z_sh.at[hi_v], add=True)
        return carry

    lax.fori_loop(0, _FULL + (wid < _REM).astype(jnp.int32), block, 0)
    plsc.subcore_barrier()

    def out(r0, nr):
        pltpu.sync_copy(z_sh.at[pl.ds(r0, nr), :], zp_hbm.at[c, pl.ds(r0, nr), :])
    _node_chunk_copy(s, out)


@functools.partial(
    pl.kernel,
    out_type=jax.ShapeDtypeStruct((_F, _E), jnp.float32),
    mesh=_mesh,
    compiler_params=pltpu.CompilerParams(needs_layout_passes=False),
    scratch_types=[
        pltpu.VMEM((_B, _D), jnp.float32),
        pltpu.VMEM((_B, _D), jnp.float32),
        pltpu.VMEM((_F, _B), jnp.float32),
        pltpu.VMEM((_F, _B), jnp.float32),
        pltpu.VMEM((_B,), jnp.int32),
        pltpu.VMEM((_B,), jnp.int32),
        pltpu.SemaphoreType.DMA,
        pltpu.SemaphoreType.DMA,
    ],
)
def _rout_sc(hpre_hbm, tpre_hbm, fv_hbm, head_hbm, tail_hbm, fv_out_hbm,
             hb_v, tb_v, fv_v, out_v, hi_v, ti_v, sem1, sem2):
    c = lax.axis_index("c")
    s = lax.axis_index("s")
    wid = s * _NC + c

    def block(i, carry):
        base = (i * _NW + wid) * _B
        pltpu.sync_copy(head_hbm.at[pl.ds(base, _B)], hi_v)
        pltpu.sync_copy(tail_hbm.at[pl.ds(base, _B)], ti_v)
        pltpu.sync_copy(fv_hbm.at[:, pl.ds(base, _B)], fv_v)
        d1 = pltpu.async_copy(hpre_hbm.at[hi_v], hb_v, sem1)
        d2 = pltpu.async_copy(tpre_hbm.at[ti_v], tb_v, sem2)
        d1.wait()
        d2.wait()
        def group(g, carry2):
            gsl = pl.ds(g * 16, 16)
            iota = lax.iota(jnp.int32, 16)
            ridx = iota + g * 16

            def dot_chunk(jj, a, f):
                for dj in range(8):
                    j = jj * 8 + dj
                    cidx = (f * _BW) + ((iota + j) & (_BW - 1))
                    hv = plsc.load_gather(hb_v, [ridx, cidx])
                    tv = plsc.load_gather(tb_v, [ridx, cidx])
                    a = a + hv * tv
                return a

            for f in range(_F):
                acc = lax.fori_loop(0, _BW // 8,
                                    functools.partial(dot_chunk, f=f),
                                    fv_v[f, gsl])
                out_v[f, gsl] = acc
            return carry2

        lax.fori_loop(0, _B // 16, group, 0)
        pltpu.sync_copy(out_v, fv_out_hbm.at[:, pl.ds(base, _B)])
        return carry

    lax.fori_loop(0, _FULL + (wid < _REM).astype(jnp.int32), block, 0)


def _prep_tc(degp_ref, ego_ref, xpre_ref, dcolt_ref):
    deg = degp_ref[0, :, :_N] + degp_ref[1, :, :_N]
    dcol = lax.rsqrt(deg).T
    dcolt_ref[...] = dcol
    xpre_ref[...] = _block_scale(ego_ref[...], dcol)


def _post_tc(zp_ref, dcolt_ref, y_ref, hpre_ref):
    z = zp_ref[0, :, :] + zp_ref[1, :, :]
    y_ref[...] = _block_scale(z, dcolt_ref[...])
    hpre_ref[...] = _block_scale(z, _block_inv_norm(z))


def _tpre_tc(ego_ref, tpre_ref):
    ego = ego_ref[...]
    tpre_ref[...] = jnp.tanh(_block_scale(ego, _block_inv_norm(ego)))


def _mean_tc(e0_ref, e1_ref, e2_ref, out_ref):
    out_ref[...] = (e0_ref[...] + e1_ref[...] + e2_ref[...]) * (1.0 / 3.0)


_prep_call = pl.pallas_call(
    _prep_tc, out_shape=[jax.ShapeDtypeStruct((_N, _D), jnp.float32),
                         jax.ShapeDtypeStruct((_N, _F), jnp.float32)])
_BN = 2000
_post_call = pl.pallas_call(
    _post_tc,
    grid=(_N // _BN,),
    in_specs=[pl.BlockSpec((_NC, _BN, _D), lambda i: (0, i, 0)),
              pl.BlockSpec((_BN, _F), lambda i: (i, 0))],
    out_specs=[pl.BlockSpec((_BN, _D), lambda i: (i, 0)),
               pl.BlockSpec((_BN, _D), lambda i: (i, 0))],
    out_shape=[jax.ShapeDtypeStruct((_N, _D), jnp.float32),
               jax.ShapeDtypeStruct((_N, _D), jnp.float32)])
_tpre_call = pl.pallas_call(
    _tpre_tc, out_shape=jax.ShapeDtypeStruct((_N, _D), jnp.float32))
_mean_call = pl.pallas_call(
    _mean_tc, out_shape=jax.ShapeDtypeStruct((_N, _D), jnp.float32))


def kernel(edge_index, user_emb, item_emb):
    head = edge_index[0].astype(jnp.int32)
    tail = edge_index[1].astype(jnp.int32)
    all_emb = jnp.concatenate([user_emb, item_emb], axis=0)
    fv = jnp.ones((_F, _E), jnp.float32)
    zeros_4n = jnp.zeros((_F, _NP), jnp.float32)
    zeros_nd = jnp.zeros((_N, _D), jnp.float32)

    embs = [all_emb]
    for l in range(_LAYER):
        ego = all_emb
        tpre = _tpre_call(ego)
        y = None
        for t in range(_ITER):
            scores, degp = _deg_sc(fv, head, zeros_4n)
            xpre, dcolt = _prep_call(degp, ego)
            zp = _spmm_sc(xpre, head, tail, scores, zeros_nd)
            y, hpre = _post_call(zp, dcolt)
            if not (l == _LAYER - 1 and t == _ITER - 1):
                fv = _rout_sc(hpre, tpre, fv, head, tail)
        all_emb = y
        embs.append(all_emb)

    out = _mean_call(embs[0], embs[1], embs[2])
    return out[:_N_USER], out[_N_USER:]


# speedup vs baseline: 12.6829x; 1.2289x over previous
import functools

import numpy as np

import jax
import jax.numpy as jnp
from jax import lax
from jax.experimental import pallas as pl
from jax.experimental.pallas import tpu as pltpu
from jax.experimental.pallas import tpu_sc as plsc

_N_USER = 5000
_N_ITEM = 5000
_N = _N_USER + _N_ITEM
_E = 320000
_D = 128
_F = 4
_BW = _D // _F
_LAYER = 2
_ITER = 2

_NC = 2
_NS = 16
_NW = _NC * _NS
_B = 128
_TOTBLK = _E // _B
_FULL = _TOTBLK // _NW
_REM = _TOTBLK - _FULL * _NW

_CHUNK = 640
_LAST_CHUNK = _N - 15 * _CHUNK

_NP = 10112
_LAST_CHUNK_P = _NP - 15 * _CHUNK

_mesh = plsc.VectorSubcoreMesh(core_axis_name="c", subcore_axis_name="s")

def _block_scale(x, col4):
    return jnp.concatenate(
        [x[:, f * _BW:(f + 1) * _BW] * col4[:, f:f + 1] for f in range(_F)],
        axis=1)


def _block_inv_norm(x):
    ss = jnp.concatenate(
        [jnp.sum(x[:, f * _BW:(f + 1) * _BW] ** 2, axis=1, keepdims=True)
         for f in range(_F)], axis=1)
    return 1.0 / jnp.maximum(jnp.sqrt(ss), 1e-12)


def _node_chunk_copy(s, fn):
    @pl.when(s < _NS - 1)
    def _():
        fn(s * _CHUNK, _CHUNK)

    @pl.when(s == _NS - 1)
    def _():
        fn((_NS - 1) * _CHUNK, _LAST_CHUNK)


def _node_chunk_copy_p(s, fn):
    @pl.when(s < _NS - 1)
    def _():
        fn(s * _CHUNK, _CHUNK)

    @pl.when(s == _NS - 1)
    def _():
        fn((_NS - 1) * _CHUNK, _LAST_CHUNK_P)


@functools.partial(
    pl.kernel,
    out_type=[
        jax.ShapeDtypeStruct((_F, _E), jnp.float32),
        jax.ShapeDtypeStruct((_NC, _F, _NP), jnp.float32),
    ],
    mesh=_mesh,
    compiler_params=pltpu.CompilerParams(needs_layout_passes=False),
    scratch_types=[
        pltpu.VMEM((_F, _B), jnp.float32),
        pltpu.VMEM((_F, _B), jnp.float32),
        pltpu.VMEM((_B,), jnp.int32),
        [pltpu.VMEM_SHARED((_NP,), jnp.float32) for _ in range(_F)],
    ],
)
def _deg_sc(fv_hbm, head_hbm, zeros_hbm, scores_hbm, degp_hbm,
            fv_v, sc_v, hi_v, deg_sh):
    c = lax.axis_index("c")
    s = lax.axis_index("s")
    wid = s * _NC + c

    def zero(r0, nr):
        for f in range(_F):
            pltpu.sync_copy(zeros_hbm.at[f, pl.ds(r0, nr)],
                            deg_sh[f].at[pl.ds(r0, nr)])
    _node_chunk_copy_p(s, zero)
    plsc.subcore_barrier()

    def block(i, carry):
        base = (i * _NW + wid) * _B
        pltpu.sync_copy(head_hbm.at[pl.ds(base, _B)], hi_v)
        pltpu.sync_copy(fv_hbm.at[:, pl.ds(base, _B)], fv_v)
        for g in range(_B // 16):
            sl = pl.ds(g * 16, 16)
            v = [fv_v[f, sl] for f in range(_F)]
            m = jnp.maximum(jnp.maximum(v[0], v[1]), jnp.maximum(v[2], v[3]))
            ex = [jnp.exp(v[f] - m) for f in range(_F)]
            inv = 1.0 / ((ex[0] + ex[1]) + (ex[2] + ex[3]))
            for f in range(_F):
                sc_v[f, sl] = ex[f] * inv
        pltpu.sync_copy(sc_v, scores_hbm.at[:, pl.ds(base, _B)])
        for f in range(_F):
            pltpu.sync_copy(sc_v.at[f], deg_sh[f].at[hi_v], add=True)
        return carry

    lax.fori_loop(0, _FULL + (wid < _REM).astype(jnp.int32), block, 0)
    plsc.subcore_barrier()

    def out(r0, nr):
        for f in range(_F):
            pltpu.sync_copy(deg_sh[f].at[pl.ds(r0, nr)],
                            degp_hbm.at[c, f, pl.ds(r0, nr)])
    _node_chunk_copy_p(s, out)


@functools.partial(
    pl.kernel,
    out_type=jax.ShapeDtypeStruct((_NC, _N, _D), jnp.float32),
    mesh=_mesh,
    compiler_params=pltpu.CompilerParams(needs_layout_passes=False),
    scratch_types=[
        [pltpu.VMEM((_B, _D), jnp.float32) for _ in range(2)],
        [pltpu.VMEM((_F, _B), jnp.float32) for _ in range(2)],
        [pltpu.VMEM((_B,), jnp.int32) for _ in range(2)],
        [pltpu.VMEM((_B,), jnp.int32) for _ in range(2)],
        pltpu.VMEM_SHARED((_N, _D), jnp.float32),
        [pltpu.SemaphoreType.DMA for _ in range(2)],
    ],
)
def _spmm_sc(xpre_hbm, head_hbm, tail_hbm, scores_hbm, zeros_hbm, zp_hbm,
             rows_v, sc_v, hi_v, ti_v, z_sh, sem):
    c = lax.axis_index("c")
    s = lax.axis_index("s")
    wid = s * _NC + c
    nblk = _FULL + (wid < _REM).astype(jnp.int32)

    def zero(r0, nr):
        pltpu.sync_copy(zeros_hbm.at[pl.ds(r0, nr), :], z_sh.at[pl.ds(r0, nr), :])
    _node_chunk_copy(s, zero)
    plsc.subcore_barrier()

    def stage_and_fire(blk, slot):
        base = (blk * _NW + wid) * _B
        pltpu.sync_copy(tail_hbm.at[pl.ds(base, _B)], ti_v[slot])
        pltpu.sync_copy(head_hbm.at[pl.ds(base, _B)], hi_v[slot])
        pltpu.sync_copy(scores_hbm.at[:, pl.ds(base, _B)], sc_v[slot])
        pltpu.async_copy(xpre_hbm.at[ti_v[slot]], rows_v[slot], sem[slot])

    stage_and_fire(0, 0)

    def pair(i2, carry):
        for b in range(2):
            blk = i2 * 2 + b

            @pl.when(blk < nblk)
            def _():
                @pl.when(blk + 1 < nblk)
                def _():
                    stage_and_fire(blk + 1, 1 - b)
                pltpu.make_async_copy(
                    xpre_hbm.at[ti_v[b]], rows_v[b], sem[b]).wait()
                for g in range(_B // 16):
                    svs = [sc_v[b][f, pl.ds(g * 16, 16)] for f in range(_F)]
                    for j in range(16):
                        e = g * 16 + j
                        for f in range(_F):
                            sf = svs[f][j]
                            for h in range(2):
                                sl = pl.ds(f * _BW + h * 16, 16)
                                rows_v[b][e, sl] = rows_v[b][e, sl] * sf
                pltpu.sync_copy(rows_v[b], z_sh.at[hi_v[b]], add=True)
        return carry

    lax.fori_loop(0, (_FULL + 2) // 2, pair, 0)
    plsc.subcore_barrier()

    def out(r0, nr):
        pltpu.sync_copy(z_sh.at[pl.ds(r0, nr), :], zp_hbm.at[c, pl.ds(r0, nr), :])
    _node_chunk_copy(s, out)


@functools.partial(
    pl.kernel,
    out_type=jax.ShapeDtypeStruct((_F, _E), jnp.float32),
    mesh=_mesh,
    compiler_params=pltpu.CompilerParams(needs_layout_passes=False),
    scratch_types=[
        [pltpu.VMEM((_B, _D), jnp.float32) for _ in range(2)],
        [pltpu.VMEM((_B, _D), jnp.float32) for _ in range(2)],
        [pltpu.VMEM((_F, _B), jnp.float32) for _ in range(2)],
        pltpu.VMEM((_F, _B), jnp.float32),
        [pltpu.VMEM((_B,), jnp.int32) for _ in range(2)],
        [pltpu.VMEM((_B,), jnp.int32) for _ in range(2)],
        [pltpu.SemaphoreType.DMA for _ in range(2)],
        [pltpu.SemaphoreType.DMA for _ in range(2)],
    ],
)
def _rout_sc(hpre_hbm, tpre_hbm, fv_hbm, head_hbm, tail_hbm, fv_out_hbm,
             hb_v, tb_v, fv_v, out_v, hi_v, ti_v, sem1, sem2):
    c = lax.axis_index("c")
    s = lax.axis_index("s")
    wid = s * _NC + c
    nblk = _FULL + (wid < _REM).astype(jnp.int32)

    def stage_and_fire(blk, slot):
        base = (blk * _NW + wid) * _B
        pltpu.sync_copy(head_hbm.at[pl.ds(base, _B)], hi_v[slot])
        pltpu.sync_copy(tail_hbm.at[pl.ds(base, _B)], ti_v[slot])
        pltpu.sync_copy(fv_hbm.at[:, pl.ds(base, _B)], fv_v[slot])
        pltpu.async_copy(hpre_hbm.at[hi_v[slot]], hb_v[slot], sem1[slot])
        pltpu.async_copy(tpre_hbm.at[ti_v[slot]], tb_v[slot], sem2[slot])

    stage_and_fire(0, 0)

    def do_block(blk, b):
        base = (blk * _NW + wid) * _B

        @pl.when(blk + 1 < nblk)
        def _():
            stage_and_fire(blk + 1, 1 - b)
        pltpu.make_async_copy(hpre_hbm.at[hi_v[b]], hb_v[b], sem1[b]).wait()
        pltpu.make_async_copy(tpre_hbm.at[ti_v[b]], tb_v[b], sem2[b]).wait()

        def group(g, carry2):
            gsl = pl.ds(g * 16, 16)
            iota = lax.iota(jnp.int32, 16)
            ridx = iota + g * 16

            def dot_chunk(jj, a, f):
                for dj in range(8):
                    j = jj * 8 + dj
                    cidx = (f * _BW) + ((iota + j) & (_BW - 1))
                    hv = plsc.load_gather(hb_v[b], [ridx, cidx])
                    tv = plsc.load_gather(tb_v[b], [ridx, cidx])
                    a = a + hv * tv
                return a

            for f in range(_F):
                acc = lax.fori_loop(0, _BW // 8,
                                    functools.partial(dot_chunk, f=f),
                                    fv_v[b][f, gsl])
                out_v[f, gsl] = acc
            return carry2

        lax.fori_loop(0, _B // 16, group, 0)
        pltpu.sync_copy(out_v, fv_out_hbm.at[:, pl.ds(base, _B)])

    def pair(i2, carry):
        for b in range(2):
            blk = i2 * 2 + b

            @pl.when(blk < nblk)
            def _():
                do_block(blk, b)
        return carry

    lax.fori_loop(0, (_FULL + 2) // 2, pair, 0)


def _prep_tc(degp_ref, ego_ref, xpre_ref, dcolt_ref):
    deg = degp_ref[0, :, :_N] + degp_ref[1, :, :_N]
    dcol = lax.rsqrt(deg).T
    dcolt_ref[...] = dcol
    xpre_ref[...] = _block_scale(ego_ref[...], dcol)


def _post_tc(zp_ref, dcolt_ref, y_ref, hpre_ref):
    z = zp_ref[0, :, :] + zp_ref[1, :, :]
    y_ref[...] = _block_scale(z, dcolt_ref[...])
    hpre_ref[...] = _block_scale(z, _block_inv_norm(z))


def _tpre_tc(ego_ref, tpre_ref):
    ego = ego_ref[...]
    tpre_ref[...] = jnp.tanh(_block_scale(ego, _block_inv_norm(ego)))


def _mean_tc(e0_ref, e1_ref, e2_ref, out_ref):
    out_ref[...] = (e0_ref[...] + e1_ref[...] + e2_ref[...]) * (1.0 / 3.0)


_prep_call = pl.pallas_call(
    _prep_tc, out_shape=[jax.ShapeDtypeStruct((_N, _D), jnp.float32),
                         jax.ShapeDtypeStruct((_N, _F), jnp.float32)])
_BN = 2000
_post_call = pl.pallas_call(
    _post_tc,
    grid=(_N // _BN,),
    in_specs=[pl.BlockSpec((_NC, _BN, _D), lambda i: (0, i, 0)),
              pl.BlockSpec((_BN, _F), lambda i: (i, 0))],
    out_specs=[pl.BlockSpec((_BN, _D), lambda i: (i, 0)),
               pl.BlockSpec((_BN, _D), lambda i: (i, 0))],
    out_shape=[jax.ShapeDtypeStruct((_N, _D), jnp.float32),
               jax.ShapeDtypeStruct((_N, _D), jnp.float32)])
_tpre_call = pl.pallas_call(
    _tpre_tc, out_shape=jax.ShapeDtypeStruct((_N, _D), jnp.float32))
_mean_call = pl.pallas_call(
    _mean_tc, out_shape=jax.ShapeDtypeStruct((_N, _D), jnp.float32))


def kernel(edge_index, user_emb, item_emb):
    head = edge_index[0].astype(jnp.int32)
    tail = edge_index[1].astype(jnp.int32)
    all_emb = jnp.concatenate([user_emb, item_emb], axis=0)
    fv = jnp.ones((_F, _E), jnp.float32)
    zeros_4n = jnp.zeros((_F, _NP), jnp.float32)
    zeros_nd = jnp.zeros((_N, _D), jnp.float32)

    embs = [all_emb]
    for l in range(_LAYER):
        ego = all_emb
        tpre = _tpre_call(ego)
        y = None
        for t in range(_ITER):
            scores, degp = _deg_sc(fv, head, zeros_4n)
            xpre, dcolt = _prep_call(degp, ego)
            zp = _spmm_sc(xpre, head, tail, scores, zeros_nd)
            y, hpre = _post_call(zp, dcolt)
            if not (l == _LAYER - 1 and t == _ITER - 1):
                fv = _rout_sc(hpre, tpre, fv, head, tail)
        all_emb = y
        embs.append(all_emb)

    out = _mean_call(embs[0], embs[1], embs[2])
    return out[:_N_USER], out[_N_USER:]


# speedup vs baseline: 13.7511x; 1.0842x over previous
import functools

import numpy as np

import jax
import jax.numpy as jnp
from jax import lax
from jax.experimental import pallas as pl
from jax.experimental.pallas import tpu as pltpu
from jax.experimental.pallas import tpu_sc as plsc

_N_USER = 5000
_N_ITEM = 5000
_N = _N_USER + _N_ITEM
_E = 320000
_D = 128
_F = 4
_BW = _D // _F
_LAYER = 2
_ITER = 2

_NC = 2
_NS = 16
_NW = _NC * _NS
_B = 128
_TOTBLK = _E // _B
_FULL = _TOTBLK // _NW
_REM = _TOTBLK - _FULL * _NW

_CHUNK = 640
_LAST_CHUNK = _N - 15 * _CHUNK

_NP = 10112
_LAST_CHUNK_P = _NP - 15 * _CHUNK

_mesh = plsc.VectorSubcoreMesh(core_axis_name="c", subcore_axis_name="s")

def _block_scale(x, col4):
    return jnp.concatenate(
        [x[:, f * _BW:(f + 1) * _BW] * col4[:, f:f + 1] for f in range(_F)],
        axis=1)


def _block_inv_norm(x):
    ss = jnp.concatenate(
        [jnp.sum(x[:, f * _BW:(f + 1) * _BW] ** 2, axis=1, keepdims=True)
         for f in range(_F)], axis=1)
    return 1.0 / jnp.maximum(jnp.sqrt(ss), 1e-12)


def _node_chunk_copy(s, fn):
    @pl.when(s < _NS - 1)
    def _():
        fn(s * _CHUNK, _CHUNK)

    @pl.when(s == _NS - 1)
    def _():
        fn((_NS - 1) * _CHUNK, _LAST_CHUNK)


def _node_chunk_copy_p(s, fn):
    @pl.when(s < _NS - 1)
    def _():
        fn(s * _CHUNK, _CHUNK)

    @pl.when(s == _NS - 1)
    def _():
        fn((_NS - 1) * _CHUNK, _LAST_CHUNK_P)


@functools.partial(
    pl.kernel,
    out_type=[
        jax.ShapeDtypeStruct((_F, _E), jnp.float32),
        jax.ShapeDtypeStruct((_NC, _F, _NP), jnp.float32),
    ],
    mesh=_mesh,
    compiler_params=pltpu.CompilerParams(needs_layout_passes=False),
    scratch_types=[
        pltpu.VMEM((_F, _B), jnp.float32),
        pltpu.VMEM((_F, _B), jnp.float32),
        pltpu.VMEM((_B,), jnp.int32),
        [pltpu.VMEM_SHARED((_NP,), jnp.float32) for _ in range(_F)],
    ],
)
def _deg_sc(fv_hbm, head_hbm, zeros_hbm, scores_hbm, degp_hbm,
            fv_v, sc_v, hi_v, deg_sh):
    c = lax.axis_index("c")
    s = lax.axis_index("s")
    wid = s * _NC + c

    def zero(r0, nr):
        for f in range(_F):
            pltpu.sync_copy(zeros_hbm.at[f, pl.ds(r0, nr)],
                            deg_sh[f].at[pl.ds(r0, nr)])
    _node_chunk_copy_p(s, zero)
    plsc.subcore_barrier()

    def block(i, carry):
        base = (i * _NW + wid) * _B
        pltpu.sync_copy(head_hbm.at[pl.ds(base, _B)], hi_v)
        pltpu.sync_copy(fv_hbm.at[:, pl.ds(base, _B)], fv_v)
        for g in range(_B // 16):
            sl = pl.ds(g * 16, 16)
            v = [fv_v[f, sl] for f in range(_F)]
            m = jnp.maximum(jnp.maximum(v[0], v[1]), jnp.maximum(v[2], v[3]))
            ex = [jnp.exp(v[f] - m) for f in range(_F)]
            inv = 1.0 / ((ex[0] + ex[1]) + (ex[2] + ex[3]))
            for f in range(_F):
                sc_v[f, sl] = ex[f] * inv
        pltpu.sync_copy(sc_v, scores_hbm.at[:, pl.ds(base, _B)])
        for f in range(_F):
            pltpu.sync_copy(sc_v.at[f], deg_sh[f].at[hi_v], add=True)
        return carry

    lax.fori_loop(0, _FULL + (wid < _REM).astype(jnp.int32), block, 0)
    plsc.subcore_barrier()

    def out(r0, nr):
        for f in range(_F):
            pltpu.sync_copy(deg_sh[f].at[pl.ds(r0, nr)],
                            degp_hbm.at[c, f, pl.ds(r0, nr)])
    _node_chunk_copy_p(s, out)


@functools.partial(
    pl.kernel,
    out_type=jax.ShapeDtypeStruct((_NC, _N, _D), jnp.float32),
    mesh=_mesh,
    compiler_params=pltpu.CompilerParams(needs_layout_passes=False),
    scratch_types=[
        [pltpu.VMEM((_B, _D), jnp.float32) for _ in range(2)],
        [pltpu.VMEM((_F, _B), jnp.float32) for _ in range(2)],
        [pltpu.VMEM((_B,), jnp.int32) for _ in range(2)],
        [pltpu.VMEM((_B,), jnp.int32) for _ in range(2)],
        pltpu.VMEM_SHARED((_N, _D), jnp.float32),
        [pltpu.SemaphoreType.DMA for _ in range(2)],
    ],
)
def _spmm_sc(xpre_hbm, head_hbm, tail_hbm, scores_hbm, zeros_hbm, zp_hbm,
             rows_v, sc_v, hi_v, ti_v, z_sh, sem):
    c = lax.axis_index("c")
    s = lax.axis_index("s")
    wid = s * _NC + c
    nblk = _FULL + (wid < _REM).astype(jnp.int32)

    def zero(r0, nr):
        pltpu.sync_copy(zeros_hbm.at[pl.ds(r0, nr), :], z_sh.at[pl.ds(r0, nr), :])
    _node_chunk_copy(s, zero)
    plsc.subcore_barrier()

    def stage_and_fire(blk, slot):
        base = (blk * _NW + wid) * _B
        pltpu.sync_copy(tail_hbm.at[pl.ds(base, _B)], ti_v[slot])
        pltpu.sync_copy(head_hbm.at[pl.ds(base, _B)], hi_v[slot])
        pltpu.sync_copy(scores_hbm.at[:, pl.ds(base, _B)], sc_v[slot])
        pltpu.async_copy(xpre_hbm.at[ti_v[slot]], rows_v[slot], sem[slot])

    stage_and_fire(0, 0)

    def pair(i2, carry):
        for b in range(2):
            blk = i2 * 2 + b

            @pl.when(blk < nblk)
            def _():
                @pl.when(blk + 1 < nblk)
                def _():
                    stage_and_fire(blk + 1, 1 - b)
                pltpu.make_async_copy(
                    xpre_hbm.at[ti_v[b]], rows_v[b], sem[b]).wait()
                for g in range(_B // 16):
                    svs = [sc_v[b][f, pl.ds(g * 16, 16)] for f in range(_F)]
                    for j in range(16):
                        e = g * 16 + j
                        for f in range(_F):
                            sf = svs[f][j]
                            for h in range(2):
                                sl = pl.ds(f * _BW + h * 16, 16)
                                rows_v[b][e, sl] = rows_v[b][e, sl] * sf
                pltpu.sync_copy(rows_v[b], z_sh.at[hi_v[b]], add=True)
        return carry

    lax.fori_loop(0, (_FULL + 2) // 2, pair, 0)
    plsc.subcore_barrier()

    def out(r0, nr):
        pltpu.sync_copy(z_sh.at[pl.ds(r0, nr), :], zp_hbm.at[c, pl.ds(r0, nr), :])
    _node_chunk_copy(s, out)


@functools.partial(
    pl.kernel,
    out_type=[
        jax.ShapeDtypeStruct((_F, _E), jnp.float32),
        jax.ShapeDtypeStruct((_F, _E), jnp.float32),
        jax.ShapeDtypeStruct((_NC, _F, _NP), jnp.float32),
    ],
    mesh=_mesh,
    compiler_params=pltpu.CompilerParams(needs_layout_passes=False),
    scratch_types=[
        [pltpu.VMEM((_B, _D), jnp.float32) for _ in range(2)],
        [pltpu.VMEM((_B, _D), jnp.float32) for _ in range(2)],
        [pltpu.VMEM((_F, _B), jnp.float32) for _ in range(2)],
        pltpu.VMEM((_F, _B), jnp.float32),
        pltpu.VMEM((_F, _B), jnp.float32),
        [pltpu.VMEM((_B,), jnp.int32) for _ in range(2)],
        [pltpu.VMEM((_B,), jnp.int32) for _ in range(2)],
        [pltpu.VMEM_SHARED((_NP,), jnp.float32) for _ in range(_F)],
        [pltpu.SemaphoreType.DMA for _ in range(2)],
        [pltpu.SemaphoreType.DMA for _ in range(2)],
    ],
)
def _rout_sc(hpre_hbm, tpre_hbm, fv_hbm, head_hbm, tail_hbm, zeros_hbm,
             fv_out_hbm, scores_hbm, degp_hbm,
             hb_v, tb_v, fv_v, out_v, sc_v, hi_v, ti_v, deg_sh,
             sem1, sem2):
    c = lax.axis_index("c")
    s = lax.axis_index("s")
    wid = s * _NC + c
    nblk = _FULL + (wid < _REM).astype(jnp.int32)

    def zero(r0, nr):
        for f in range(_F):
            pltpu.sync_copy(zeros_hbm.at[f, pl.ds(r0, nr)],
                            deg_sh[f].at[pl.ds(r0, nr)])
    _node_chunk_copy_p(s, zero)
    plsc.subcore_barrier()

    def stage_and_fire(blk, slot):
        base = (blk * _NW + wid) * _B
        pltpu.sync_copy(head_hbm.at[pl.ds(base, _B)], hi_v[slot])
        pltpu.sync_copy(tail_hbm.at[pl.ds(base, _B)], ti_v[slot])
        pltpu.sync_copy(fv_hbm.at[:, pl.ds(base, _B)], fv_v[slot])
        pltpu.async_copy(hpre_hbm.at[hi_v[slot]], hb_v[slot], sem1[slot])
        pltpu.async_copy(tpre_hbm.at[ti_v[slot]], tb_v[slot], sem2[slot])

    stage_and_fire(0, 0)

    def do_block(blk, b):
        base = (blk * _NW + wid) * _B

        @pl.when(blk + 1 < nblk)
        def _():
            stage_and_fire(blk + 1, 1 - b)
        pltpu.make_async_copy(hpre_hbm.at[hi_v[b]], hb_v[b], sem1[b]).wait()
        pltpu.make_async_copy(tpre_hbm.at[ti_v[b]], tb_v[b], sem2[b]).wait()

        def group(g, carry2):
            gsl = pl.ds(g * 16, 16)
            iota = lax.iota(jnp.int32, 16)
            ridx = iota + g * 16

            def dot_chunk(jj, a, f):
                for dj in range(8):
                    j = jj * 8 + dj
                    cidx = (f * _BW) + ((iota + j) & (_BW - 1))
                    hv = plsc.load_gather(hb_v[b], [ridx, cidx])
                    tv = plsc.load_gather(tb_v[b], [ridx, cidx])
                    a = a + hv * tv
                return a

            acc = [None] * _F
            for f in range(_F):
                acc[f] = lax.fori_loop(0, _BW // 8,
                                       functools.partial(dot_chunk, f=f),
                                       fv_v[b][f, gsl])
                out_v[f, gsl] = acc[f]
            m = jnp.maximum(jnp.maximum(acc[0], acc[1]),
                            jnp.maximum(acc[2], acc[3]))
            ex = [jnp.exp(acc[f] - m) for f in range(_F)]
            inv = 1.0 / ((ex[0] + ex[1]) + (ex[2] + ex[3]))
            for f in range(_F):
                sc_v[f, gsl] = ex[f] * inv
            return carry2

        lax.fori_loop(0, _B // 16, group, 0)
        pltpu.sync_copy(out_v, fv_out_hbm.at[:, pl.ds(base, _B)])
        pltpu.sync_copy(sc_v, scores_hbm.at[:, pl.ds(base, _B)])
        for f in range(_F):
            pltpu.sync_copy(sc_v.at[f], deg_sh[f].at[hi_v[b]], add=True)

    def pair(i2, carry):
        for b in range(2):
            blk = i2 * 2 + b

            @pl.when(blk < nblk)
            def _():
                do_block(blk, b)
        return carry

    lax.fori_loop(0, (_FULL + 2) // 2, pair, 0)
    plsc.subcore_barrier()

    def out(r0, nr):
        for f in range(_F):
            pltpu.sync_copy(deg_sh[f].at[pl.ds(r0, nr)],
                            degp_hbm.at[c, f, pl.ds(r0, nr)])
    _node_chunk_copy_p(s, out)


def _prep_tc(degp_ref, ego_ref, xpre_ref, dcolt_ref):
    deg = degp_ref[0, :, :_N] + degp_ref[1, :, :_N]
    dcol = lax.rsqrt(deg).T
    dcolt_ref[...] = dcol
    xpre_ref[...] = _block_scale(ego_ref[...], dcol)


def _post_tc(zp_ref, dcolt_ref, y_ref, hpre_ref):
    z = zp_ref[0, :, :] + zp_ref[1, :, :]
    y_ref[...] = _block_scale(z, dcolt_ref[...])
    hpre_ref[...] = _block_scale(z, _block_inv_norm(z))


def _tpre_tc(ego_ref, tpre_ref):
    ego = ego_ref[...]
    tpre_ref[...] = jnp.tanh(_block_scale(ego, _block_inv_norm(ego)))


def _mean_tc(e0_ref, e1_ref, e2_ref, out_ref):
    out_ref[...] = (e0_ref[...] + e1_ref[...] + e2_ref[...]) * (1.0 / 3.0)


_prep_call = pl.pallas_call(
    _prep_tc, out_shape=[jax.ShapeDtypeStruct((_N, _D), jnp.float32),
                         jax.ShapeDtypeStruct((_N, _F), jnp.float32)])
_BN = 2000
_post_call = pl.pallas_call(
    _post_tc,
    grid=(_N // _BN,),
    in_specs=[pl.BlockSpec((_NC, _BN, _D), lambda i: (0, i, 0)),
              pl.BlockSpec((_BN, _F), lambda i: (i, 0))],
    out_specs=[pl.BlockSpec((_BN, _D), lambda i: (i, 0)),
               pl.BlockSpec((_BN, _D), lambda i: (i, 0))],
    out_shape=[jax.ShapeDtypeStruct((_N, _D), jnp.float32),
               jax.ShapeDtypeStruct((_N, _D), jnp.float32)])
_tpre_call = pl.pallas_call(
    _tpre_tc, out_shape=jax.ShapeDtypeStruct((_N, _D), jnp.float32))
_mean_call = pl.pallas_call(
    _mean_tc, out_shape=jax.ShapeDtypeStruct((_N, _D), jnp.float32))


def kernel(edge_index, user_emb, item_emb):
    head = edge_index[0].astype(jnp.int32)
    tail = edge_index[1].astype(jnp.int32)
    all_emb = jnp.concatenate([user_emb, item_emb], axis=0)
    fv = jnp.ones((_F, _E), jnp.float32)
    zeros_4n = jnp.zeros((_F, _NP), jnp.float32)
    zeros_nd = jnp.zeros((_N, _D), jnp.float32)

    embs = [all_emb]
    scores = degp = None
    for l in range(_LAYER):
        ego = all_emb
        tpre = _tpre_call(ego)
        y = None
        for t in range(_ITER):
            if scores is None:
                scores, degp = _deg_sc(fv, head, zeros_4n)
            xpre, dcolt = _prep_call(degp, ego)
            zp = _spmm_sc(xpre, head, tail, scores, zeros_nd)
            y, hpre = _post_call(zp, dcolt)
            if not (l == _LAYER - 1 and t == _ITER - 1):
                fv, scores, degp = _rout_sc(hpre, tpre, fv, head, tail,
                                            zeros_4n)
        all_emb = y
        embs.append(all_emb)

    out = _mean_call(embs[0], embs[1], embs[2])
    return out[:_N_USER], out[_N_USER:]
